# baseline (speedup 1.0000x reference)
import functools
import math

import jax
import jax.numpy as jnp
from jax import lax
from jax.experimental import pallas as pl
from jax.experimental.pallas import tpu as pltpu
from jax.experimental.pallas import tpu_sc as plsc

NE = 8
TOPK = 2
CAP_F = 1.25
LANES = 128
NC = 2
NS = 16
NW = NC * NS
SCL = 16


def _router_body(cap, T, x_ref, rw_ref, ltri_ref, dsts_ref, dstr_ref,
                 w0_ref, w1_ref, aux_ref, z_ref):
    f32 = jnp.float32
    x = x_ref[...]
    logits = jnp.dot(x, rw_ref[...], preferred_element_type=f32)
    lane = lax.broadcasted_iota(jnp.int32, (T, LANES), 1)
    valid_lane = lane < NE
    lm = jnp.where(valid_lane, logits, -1e30)
    mx = jnp.max(lm, axis=1, keepdims=True)
    ex = jnp.where(valid_lane, jnp.exp(lm - mx), 0.0)
    se = jnp.sum(ex, axis=1, keepdims=True)
    probs = ex / se

    m0 = jnp.max(probs, axis=1, keepdims=True)
    i0 = jnp.min(jnp.where((probs == m0) & valid_lane, lane, NE),
                 axis=1, keepdims=True)
    sel0 = lane == i0
    p2 = jnp.where(sel0 | ~valid_lane, -1.0, probs)
    m1 = jnp.max(p2, axis=1, keepdims=True)
    i1 = jnp.min(jnp.where((p2 == m1) & valid_lane, lane, NE),
                 axis=1, keepdims=True)
    sel1 = lane == i1

    B = 128
    L = ltri_ref[...]
    mask0 = sel0.astype(f32)
    mask1 = sel1.astype(f32)
    carry0 = jnp.zeros((1, LANES), f32)
    carry1 = jnp.zeros((1, LANES), f32)
    blocks0 = []
    blocks1 = []
    for b in range(T // B):
        mb0 = mask0[b * B:(b + 1) * B, :]
        mb1 = mask1[b * B:(b + 1) * B, :]
        cb0 = jnp.dot(L, mb0, preferred_element_type=f32) + carry0
        cb1 = jnp.dot(L, mb1, preferred_element_type=f32) + carry1
        carry0 = cb0[B - 1:B, :]
        carry1 = cb1[B - 1:B, :]
        blocks0.append(cb0)
        blocks1.append(cb1)
    c0 = jnp.concatenate(blocks0, axis=0)
    c1 = jnp.concatenate(blocks1, axis=0)
    tot0 = carry0
    tot1 = carry1

    s0 = jnp.sum(jnp.where(sel0, c0 - 1.0, 0.0), axis=1, keepdims=True)
    s1 = jnp.sum(jnp.where(sel1, c1 - 1.0 + tot0, 0.0), axis=1, keepdims=True)
    s0i = s0.astype(jnp.int32)
    s1i = s1.astype(jnp.int32)
    v0 = s0i < cap
    v1 = s1i < cap
    tv0 = jnp.sum(jnp.where(sel0, probs, 0.0), axis=1, keepdims=True)
    tv1 = jnp.sum(jnp.where(sel1, probs, 0.0), axis=1, keepdims=True)

    trash = NE * cap
    d0 = jnp.where(v0, i0 * cap + s0i, trash)
    d1 = jnp.where(v1, i1 * cap + s1i, trash)
    dsts_ref[:, 0:1] = d0
    dsts_ref[:, 1:2] = d1
    dstr_ref[:, 0:1] = jnp.where(v0, d0, 0)
    dstr_ref[:, 1:2] = jnp.where(v1, d1, 0)
    ones16 = jnp.ones((1, SCL), f32)
    w0_ref[...] = jnp.where(v0, tv0, 0.0) * ones16
    w1_ref[...] = jnp.where(v1, tv1, 0.0) * ones16

    count = jnp.minimum(jnp.float32(cap), tot0 + tot1)
    mean_prob = jnp.sum(probs, axis=0, keepdims=True) / T
    aux_ref[...] = NE * jnp.sum(mean_prob * count, keepdims=True) / T
    lse = mx + jnp.log(se)
    z_ref[...] = jnp.sum(lse * lse, keepdims=True).reshape(1, 1) / T


def _router_call(x_flat, rw_pad, ltri, cap):
    T = x_flat.shape[0]
    f32 = jnp.float32
    return pl.pallas_call(
        functools.partial(_router_body, cap, T),
        out_shape=[
            jax.ShapeDtypeStruct((T, 2), jnp.int32),
            jax.ShapeDtypeStruct((T, 2), jnp.int32),
            jax.ShapeDtypeStruct((T, SCL), f32),
            jax.ShapeDtypeStruct((T, SCL), f32),
            jax.ShapeDtypeStruct((1, 1), f32),
            jax.ShapeDtypeStruct((1, 1), f32),
        ],
    )(x_flat, rw_pad, ltri)


def _dispatch_body(T, H, ch, x_hbm, dst_hbm, disp_hbm, xbuf, idx0, idx1, sem):
    tpw = T // NW
    wid = lax.axis_index("s") * NC + lax.axis_index("c")
    base = wid * tpw

    def chunk(ci, carry):
        tb = base + ci * ch
        pltpu.sync_copy(dst_hbm.at[pl.ds(tb, ch)], idx0)
        pltpu.sync_copy(dst_hbm.at[pl.ds(T + tb, ch)], idx1)
        pltpu.sync_copy(x_hbm.at[pl.ds(tb, ch)], xbuf)
        c0 = pltpu.async_copy(xbuf, disp_hbm.at[idx0], sem)
        c1 = pltpu.async_copy(xbuf, disp_hbm.at[idx1], sem)
        c0.wait()
        c1.wait()
        return carry

    lax.fori_loop(0, tpw // ch, chunk, 0)


def _dispatch_call(x_flat, dsts_flat, cap):
    T, H = x_flat.shape
    ch = 32
    mesh = plsc.VectorSubcoreMesh(core_axis_name="c", subcore_axis_name="s",
                                  num_cores=NC, num_subcores=NS)
    return pl.kernel(
        functools.partial(_dispatch_body, T, H, ch),
        out_type=jax.ShapeDtypeStruct((NE * cap + 8, H), jnp.float32),
        mesh=mesh,
        scratch_types=[
            pltpu.VMEM((ch, H), jnp.float32),
            pltpu.VMEM((ch,), jnp.int32),
            pltpu.VMEM((ch,), jnp.int32),
            pltpu.SemaphoreType.DMA,
        ],
    )(x_flat, dsts_flat)


def _ffn_body(x_ref, gw_ref, gb_ref, uw_ref, ub_ref, dw_ref, db_ref, out_ref):
    i = pl.program_id(1)
    f32 = jnp.float32
    x = x_ref[...].astype(jnp.bfloat16)
    g = jnp.dot(x, gw_ref[0], preferred_element_type=f32) + gb_ref[0]
    u = jnp.dot(x, uw_ref[0], preferred_element_type=f32) + ub_ref[0]
    h = g * (1.0 / (1.0 + jnp.exp(-g))) * u
    part = jnp.dot(h.astype(jnp.bfloat16), dw_ref[0], preferred_element_type=f32)

    @pl.when(i == 0)
    def _():
        out_ref[...] = part + db_ref[0]

    @pl.when(i > 0)
    def _():
        out_ref[...] = out_ref[...] + part


def _ffn_call(disp, gate_w, gate_b, up_w, up_b, down_w, down_b, cap):
    H = disp.shape[1]
    I = gate_w.shape[2]
    NI = 8
    TI = I // NI
    f32 = jnp.float32
    return pl.pallas_call(
        _ffn_body,
        grid=(NE, NI),
        in_specs=[
            pl.BlockSpec((cap, H), lambda e, i: (e, 0)),
            pl.BlockSpec((1, H, TI), lambda e, i: (e, 0, i)),
            pl.BlockSpec((1, 1, TI), lambda e, i: (e, 0, i)),
            pl.BlockSpec((1, H, TI), lambda e, i: (e, 0, i)),
            pl.BlockSpec((1, 1, TI), lambda e, i: (e, 0, i)),
            pl.BlockSpec((1, TI, H), lambda e, i: (e, i, 0)),
            pl.BlockSpec((1, 1, H), lambda e, i: (e, 0, 0)),
        ],
        out_specs=pl.BlockSpec((cap, H), lambda e, i: (e, 0)),
        out_shape=jax.ShapeDtypeStruct((NE * cap, H), f32),
    )(disp, gate_w, gate_b[:, None, :], up_w, up_b[:, None, :],
      down_w, down_b[:, None, :])


def _combine_body(T, H, ch, ffn_hbm, dstr_hbm, w0_hbm, w1_hbm, out_hbm,
                  b0, b1, ob, i0b, i1b, w0b, w1b, sem):
    tpw = T // NW
    wid = lax.axis_index("s") * NC + lax.axis_index("c")
    base = wid * tpw

    def chunk(ci, carry):
        tb = base + ci * ch
        pltpu.sync_copy(dstr_hbm.at[pl.ds(tb, ch)], i0b)
        pltpu.sync_copy(dstr_hbm.at[pl.ds(T + tb, ch)], i1b)
        pltpu.sync_copy(w0_hbm.at[pl.ds(tb, ch)], w0b)
        pltpu.sync_copy(w1_hbm.at[pl.ds(tb, ch)], w1b)
        g0 = pltpu.async_copy(ffn_hbm.at[i0b], b0, sem)
        g1 = pltpu.async_copy(ffn_hbm.at[i1b], b1, sem)
        g0.wait()
        g1.wait()

        def row(i, rc):
            w0v = w0b[i, :]
            w1v = w1b[i, :]

            def grp(j, gc):
                sl = pl.ds(j * SCL, SCL)
                ob[i, sl] = b0[i, sl] * w0v + b1[i, sl] * w1v
                return gc

            lax.fori_loop(0, H // SCL, grp, 0)
            return rc

        lax.fori_loop(0, ch, row, 0)
        pltpu.sync_copy(ob, out_hbm.at[pl.ds(tb, ch)])
        return carry

    lax.fori_loop(0, tpw // ch, chunk, 0)


def _combine_call(ffn, dstr_flat, w0x, w1x, T):
    H = ffn.shape[1]
    ch = 32
    mesh = plsc.VectorSubcoreMesh(core_axis_name="c", subcore_axis_name="s",
                                  num_cores=NC, num_subcores=NS)
    return pl.kernel(
        functools.partial(_combine_body, T, H, ch),
        out_type=jax.ShapeDtypeStruct((T, H), jnp.float32),
        mesh=mesh,
        scratch_types=[
            pltpu.VMEM((ch, H), jnp.float32),
            pltpu.VMEM((ch, H), jnp.float32),
            pltpu.VMEM((ch, H), jnp.float32),
            pltpu.VMEM((ch,), jnp.int32),
            pltpu.VMEM((ch,), jnp.int32),
            pltpu.VMEM((ch, SCL), jnp.float32),
            pltpu.VMEM((ch, SCL), jnp.float32),
            pltpu.SemaphoreType.DMA,
        ],
    )(ffn, dstr_flat, w0x, w1x)


def kernel(x, router_w, gate_w, gate_b, up_w, up_b, down_w, down_b):
    bsz, seq, H = x.shape
    T = bsz * seq
    cap = max(1, math.ceil(CAP_F * T / NE))
    x_flat = x.reshape(T, H)

    rw_pad = jnp.pad(router_w, ((0, 0), (0, LANES - NE)))
    ltri = jnp.tril(jnp.ones((128, 128), jnp.float32))

    dsts, dstr, w0x, w1x, aux, z = _router_call(x_flat, rw_pad, ltri, cap)
    dsts_flat = dsts.T.reshape(-1)
    dstr_flat = dstr.T.reshape(-1)

    disp = _dispatch_call(x_flat, dsts_flat, cap)
    bf16 = jnp.bfloat16
    ffn = _ffn_call(disp, gate_w.astype(bf16), gate_b,
                    up_w.astype(bf16), up_b, down_w.astype(bf16), down_b, cap)
    out = _combine_call(ffn, dstr_flat, w0x, w1x, T)

    return (out.reshape(bsz, seq, H), aux.reshape(()), z.reshape(()))


# speedup vs baseline: 1.3986x; 1.3986x over previous
import functools
import math

import jax
import jax.numpy as jnp
from jax import lax
from jax.experimental import pallas as pl
from jax.experimental.pallas import tpu as pltpu
from jax.experimental.pallas import tpu_sc as plsc

NE = 8
TOPK = 2
CAP_F = 1.25
LANES = 128
NC = 2
NS = 16
NW = NC * NS
SCL = 16


def _router_body(cap, T, x_ref, rw_ref, ltri_ref, dsts_ref, dstr_ref,
                 w0_ref, w1_ref, aux_ref, z_ref):
    f32 = jnp.float32
    x = x_ref[...]
    logits = jnp.dot(x, rw_ref[...], preferred_element_type=f32)
    lane = lax.broadcasted_iota(jnp.int32, (T, LANES), 1)
    valid_lane = lane < NE
    lm = jnp.where(valid_lane, logits, -1e30)
    mx = jnp.max(lm, axis=1, keepdims=True)
    ex = jnp.where(valid_lane, jnp.exp(lm - mx), 0.0)
    se = jnp.sum(ex, axis=1, keepdims=True)
    probs = ex / se

    m0 = jnp.max(probs, axis=1, keepdims=True)
    i0 = jnp.min(jnp.where((probs == m0) & valid_lane, lane, NE),
                 axis=1, keepdims=True)
    sel0 = lane == i0
    p2 = jnp.where(sel0 | ~valid_lane, -1.0, probs)
    m1 = jnp.max(p2, axis=1, keepdims=True)
    i1 = jnp.min(jnp.where((p2 == m1) & valid_lane, lane, NE),
                 axis=1, keepdims=True)
    sel1 = lane == i1

    B = 128
    L = ltri_ref[...]
    mask0 = sel0.astype(f32)
    mask1 = sel1.astype(f32)
    carry0 = jnp.zeros((1, LANES), f32)
    carry1 = jnp.zeros((1, LANES), f32)
    blocks0 = []
    blocks1 = []
    for b in range(T // B):
        mb0 = mask0[b * B:(b + 1) * B, :]
        mb1 = mask1[b * B:(b + 1) * B, :]
        cb0 = jnp.dot(L, mb0, preferred_element_type=f32) + carry0
        cb1 = jnp.dot(L, mb1, preferred_element_type=f32) + carry1
        carry0 = cb0[B - 1:B, :]
        carry1 = cb1[B - 1:B, :]
        blocks0.append(cb0)
        blocks1.append(cb1)
    c0 = jnp.concatenate(blocks0, axis=0)
    c1 = jnp.concatenate(blocks1, axis=0)
    tot0 = carry0
    tot1 = carry1

    s0 = jnp.sum(jnp.where(sel0, c0 - 1.0, 0.0), axis=1, keepdims=True)
    s1 = jnp.sum(jnp.where(sel1, c1 - 1.0 + tot0, 0.0), axis=1, keepdims=True)
    s0i = s0.astype(jnp.int32)
    s1i = s1.astype(jnp.int32)
    v0 = s0i < cap
    v1 = s1i < cap
    tv0 = jnp.sum(jnp.where(sel0, probs, 0.0), axis=1, keepdims=True)
    tv1 = jnp.sum(jnp.where(sel1, probs, 0.0), axis=1, keepdims=True)

    trash = NE * cap
    d0 = jnp.where(v0, i0 * cap + s0i, trash)
    d1 = jnp.where(v1, i1 * cap + s1i, trash)
    dsts_ref[:, 0:1] = d0
    dsts_ref[:, 1:2] = d1
    dstr_ref[:, 0:1] = jnp.where(v0, d0, 0)
    dstr_ref[:, 1:2] = jnp.where(v1, d1, 0)
    ones16 = jnp.ones((1, SCL), f32)
    w0_ref[...] = jnp.where(v0, tv0, 0.0) * ones16
    w1_ref[...] = jnp.where(v1, tv1, 0.0) * ones16

    count = jnp.minimum(jnp.float32(cap), tot0 + tot1)
    mean_prob = jnp.sum(probs, axis=0, keepdims=True) / T
    aux_ref[...] = NE * jnp.sum(mean_prob * count, keepdims=True) / T
    lse = mx + jnp.log(se)
    z_ref[...] = jnp.sum(lse * lse, keepdims=True).reshape(1, 1) / T


def _router_call(x_flat, rw_pad, ltri, cap):
    T = x_flat.shape[0]
    f32 = jnp.float32
    return pl.pallas_call(
        functools.partial(_router_body, cap, T),
        out_shape=[
            jax.ShapeDtypeStruct((T, 2), jnp.int32),
            jax.ShapeDtypeStruct((T, 2), jnp.int32),
            jax.ShapeDtypeStruct((T, SCL), f32),
            jax.ShapeDtypeStruct((T, SCL), f32),
            jax.ShapeDtypeStruct((1, 1), f32),
            jax.ShapeDtypeStruct((1, 1), f32),
        ],
    )(x_flat, rw_pad, ltri)


DCH = 32
NBUF = 3


def _dispatch_body(T, H, x_hbm, dst_hbm, disp_hbm, idxb, b0, b1, b2,
                   semx, sems):
    tpw = T // NW
    nch = tpw // DCH
    wid = lax.axis_index("s") * NC + lax.axis_index("c")
    base = wid * tpw
    bufs = [b0, b1, b2]

    pltpu.sync_copy(dst_hbm.at[wid], idxb)

    def load(ci):
        return pltpu.async_copy(x_hbm.at[pl.ds(base + ci * DCH, DCH)],
                                bufs[ci % NBUF], semx)

    loads = [None] * nch
    scats = [None] * nch
    issued = min(NBUF, nch)
    for ci in range(issued):
        loads[ci] = load(ci)
    drained = set()
    for ci in range(nch):
        loads[ci].wait()
        b = bufs[ci % NBUF]
        scats[ci] = (pltpu.async_copy(b, disp_hbm.at[idxb.at[ci]], sems),
                     pltpu.async_copy(b, disp_hbm.at[idxb.at[nch + ci]], sems))
        if issued < nch and ci >= 1:
            k = issued
            for s in scats[k - NBUF]:
                s.wait()
            drained.add(k - NBUF)
            loads[k] = load(k)
            issued += 1
    for ci in range(nch):
        if ci not in drained:
            for s in scats[ci]:
                s.wait()


def _dispatch_call(x_flat, dsts_tiled, cap):
    T, H = x_flat.shape
    nch = T // NW // DCH
    mesh = plsc.VectorSubcoreMesh(core_axis_name="c", subcore_axis_name="s",
                                  num_cores=NC, num_subcores=NS)
    return pl.kernel(
        functools.partial(_dispatch_body, T, H),
        out_type=jax.ShapeDtypeStruct((NE * cap + 8, H), jnp.float32),
        mesh=mesh,
        scratch_types=[
            pltpu.VMEM((2 * nch, DCH), jnp.int32),
            pltpu.VMEM((DCH, H), jnp.float32),
            pltpu.VMEM((DCH, H), jnp.float32),
            pltpu.VMEM((DCH, H), jnp.float32),
            pltpu.SemaphoreType.DMA,
            pltpu.SemaphoreType.DMA,
        ],
    )(x_flat, dsts_tiled)


def _ffn_body(x_ref, gw_ref, gb_ref, uw_ref, ub_ref, dw_ref, db_ref, out_ref):
    i = pl.program_id(1)
    f32 = jnp.float32
    x = x_ref[...]
    g = jnp.dot(x, gw_ref[0], preferred_element_type=f32) + gb_ref[0]
    u = jnp.dot(x, uw_ref[0], preferred_element_type=f32) + ub_ref[0]
    h = g * (1.0 / (1.0 + jnp.exp(-g))) * u
    part = jnp.dot(h, dw_ref[0], preferred_element_type=f32)

    @pl.when(i == 0)
    def _():
        out_ref[...] = part + db_ref[0]

    @pl.when(i > 0)
    def _():
        out_ref[...] = out_ref[...] + part


def _ffn_call(disp, gate_w, gate_b, up_w, up_b, down_w, down_b, cap):
    H = disp.shape[1]
    I = gate_w.shape[2]
    NI = 8
    TI = I // NI
    f32 = jnp.float32
    return pl.pallas_call(
        _ffn_body,
        grid=(NE, NI),
        in_specs=[
            pl.BlockSpec((cap, H), lambda e, i: (e, 0)),
            pl.BlockSpec((1, H, TI), lambda e, i: (e, 0, i)),
            pl.BlockSpec((1, 1, TI), lambda e, i: (e, 0, i)),
            pl.BlockSpec((1, H, TI), lambda e, i: (e, 0, i)),
            pl.BlockSpec((1, 1, TI), lambda e, i: (e, 0, i)),
            pl.BlockSpec((1, TI, H), lambda e, i: (e, i, 0)),
            pl.BlockSpec((1, 1, H), lambda e, i: (e, 0, 0)),
        ],
        out_specs=pl.BlockSpec((cap, H), lambda e, i: (e, 0)),
        out_shape=jax.ShapeDtypeStruct((NE * cap, H), f32),
    )(disp, gate_w, gate_b[:, None, :], up_w, up_b[:, None, :],
      down_w, down_b[:, None, :])


CCH = 16


def _combine_body(T, H, ffn_hbm, dstr_hbm, w0_hbm, w1_hbm, out_hbm,
                  idxb, w0b, w1b, a0, a1, ao, c0, c1, co, semg, semo):
    tpw = T // NW
    nch = tpw // CCH
    wid = lax.axis_index("s") * NC + lax.axis_index("c")
    base = wid * tpw
    sets = [(a0, a1, ao), (c0, c1, co)]

    pltpu.sync_copy(dstr_hbm.at[wid], idxb)
    pltpu.sync_copy(w0_hbm.at[pl.ds(base * SCL, tpw * SCL)], w0b)
    pltpu.sync_copy(w1_hbm.at[pl.ds(base * SCL, tpw * SCL)], w1b)

    def gathers(ci):
        s0, s1, _ = sets[ci % 2]
        return (pltpu.async_copy(ffn_hbm.at[idxb.at[ci]], s0, semg),
                pltpu.async_copy(ffn_hbm.at[idxb.at[nch + ci]], s1, semg))

    gat = [None] * nch
    sto = [None] * nch
    for ci in range(min(2, nch)):
        gat[ci] = gathers(ci)

    for ci in range(nch):
        s0, s1, so = sets[ci % 2]
        for g in gat[ci]:
            g.wait()
        if ci >= 2:
            sto[ci - 2].wait()

        def row(i, rc):
            w0v = w0b[pl.ds((ci * CCH + i) * SCL, SCL)]
            w1v = w1b[pl.ds((ci * CCH + i) * SCL, SCL)]
            for j in range(H // SCL):
                sl = pl.ds(j * SCL, SCL)
                so[i, sl] = s0[i, sl] * w0v + s1[i, sl] * w1v
            return rc

        lax.fori_loop(0, CCH, row, 0)
        sto[ci] = pltpu.async_copy(
            so, out_hbm.at[pl.ds(base + ci * CCH, CCH)], semo)
        if ci + 2 < nch:
            gat[ci + 2] = gathers(ci + 2)

    for ci in range(max(0, nch - 2), nch):
        sto[ci].wait()


def _combine_call(ffn, dstr_tiled, w0x, w1x, T):
    H = ffn.shape[1]
    tpw = T // NW
    nch = tpw // CCH
    mesh = plsc.VectorSubcoreMesh(core_axis_name="c", subcore_axis_name="s",
                                  num_cores=NC, num_subcores=NS)
    return pl.kernel(
        functools.partial(_combine_body, T, H),
        out_type=jax.ShapeDtypeStruct((T, H), jnp.float32),
        mesh=mesh,
        scratch_types=[
            pltpu.VMEM((2 * nch, CCH), jnp.int32),
            pltpu.VMEM((tpw * SCL,), jnp.float32),
            pltpu.VMEM((tpw * SCL,), jnp.float32),
            pltpu.VMEM((CCH, H), jnp.float32),
            pltpu.VMEM((CCH, H), jnp.float32),
            pltpu.VMEM((CCH, H), jnp.float32),
            pltpu.VMEM((CCH, H), jnp.float32),
            pltpu.VMEM((CCH, H), jnp.float32),
            pltpu.VMEM((CCH, H), jnp.float32),
            pltpu.SemaphoreType.DMA,
            pltpu.SemaphoreType.DMA,
        ],
    )(ffn, dstr_tiled, w0x, w1x)


def kernel(x, router_w, gate_w, gate_b, up_w, up_b, down_w, down_b):
    bsz, seq, H = x.shape
    T = bsz * seq
    cap = max(1, math.ceil(CAP_F * T / NE))
    x_flat = x.reshape(T, H)

    rw_pad = jnp.pad(router_w, ((0, 0), (0, LANES - NE)))
    ltri = jnp.tril(jnp.ones((128, 128), jnp.float32))

    dsts, dstr, w0x, w1x, aux, z = _router_call(x_flat, rw_pad, ltri, cap)

    def tile_idx(d, ch):
        nch = T // NW // ch
        a = d.T.reshape(2, NW, nch, ch)
        return jnp.transpose(a, (1, 0, 2, 3)).reshape(NW, 2 * nch, ch)

    disp = _dispatch_call(x_flat, tile_idx(dsts, DCH), cap)
    ffn = _ffn_call(disp, gate_w, gate_b, up_w, up_b, down_w, down_b, cap)
    out = _combine_call(ffn, tile_idx(dstr, CCH), w0x.reshape(-1),
                        w1x.reshape(-1), T)

    return (out.reshape(bsz, seq, H), aux.reshape(()), z.reshape(()))
